# baseline (speedup 1.0000x reference)
import jax
import jax.numpy as jnp
from jax.experimental import pallas as pl
from jax.experimental.pallas import tpu as pltpu

S = 64.0
M = 0.4

_BR = 3072


def _mk_body(base):
    def _cosface_block(lab_ref, x_ref, o_ref):
        i = pl.program_id(0)
        lab = lab_ref[...]
        row = jax.lax.broadcasted_iota(jnp.int32, x_ref.shape, 0) + base + i * _BR
        hit = row == lab
        x = x_ref[...]
        o_ref[...] = x * S - (M * S) * hit.astype(jnp.float32)
    return _cosface_block


def _piece(lt, lab2, base, rows, B, dtype):
    nb0 = base // _BR
    return pl.pallas_call(
        _mk_body(base),
        grid=(pl.cdiv(rows, _BR),),
        in_specs=[
            pl.BlockSpec((1, B), lambda i: (0, 0)),
            pl.BlockSpec((_BR, B), lambda i: (i + nb0, 0)),
        ],
        out_specs=pl.BlockSpec((_BR, B), lambda i: (i, 0)),
        out_shape=jax.ShapeDtypeStruct((rows, B), dtype),
        compiler_params=pltpu.CompilerParams(
            dimension_semantics=("arbitrary",),
        ),
    )(lab2, lt)


def kernel(logits, labels):
    B, C = logits.shape
    lt = logits.T
    lab2 = labels.reshape(1, B)
    split = 17 * _BR
    a = _piece(lt, lab2, 0, split, B, logits.dtype)
    b = _piece(lt, lab2, split, C - split, B, logits.dtype)
    return jnp.concatenate([a, b], axis=0).T


# speedup vs baseline: 2.0125x; 2.0125x over previous
import jax
import jax.numpy as jnp
from jax.experimental import pallas as pl
from jax.experimental.pallas import tpu as pltpu

S = 64.0
M = 0.4

_BR = 3072


def _cosface_block(lab_ref, x_ref, o_ref):
    i = pl.program_id(0)
    lab = lab_ref[...]
    row = jax.lax.broadcasted_iota(jnp.int32, x_ref.shape, 0) + i * _BR
    hit = row == lab
    x = x_ref[...]
    o_ref[...] = x * S - (M * S) * hit.astype(jnp.float32)


def kernel(logits, labels):
    B, C = logits.shape
    lt = logits.T
    lab2 = labels.reshape(1, B)
    out_t = pl.pallas_call(
        _cosface_block,
        grid=(pl.cdiv(C, _BR),),
        in_specs=[
            pl.BlockSpec((1, B), lambda i: (0, 0)),
            pl.BlockSpec((_BR, B), lambda i: (i, 0)),
        ],
        out_specs=pl.BlockSpec((_BR, B), lambda i: (i, 0)),
        out_shape=jax.ShapeDtypeStruct((C, B), logits.dtype),
        compiler_params=pltpu.CompilerParams(
            dimension_semantics=("arbitrary",),
        ),
    )(lab2, lt)
    return out_t.T
